# Initial kernel scaffold; baseline (speedup 1.0000x reference)
import functools

import jax
import jax.numpy as jnp
from jax import lax
from jax.experimental import pallas as pl
from jax.experimental.pallas import tpu as pltpu
from jax.experimental.pallas import tpu_sc as plsc

M = 100000
D = 128
B = 16384
NC, NS = 2, 16
NW = NC * NS
BPW = B // NW
SUB = 128
NSUB = BPW // SUB
ROWS2D = B // SUB
RPW = ROWS2D // NW
DEDUP_WID = NW - 1
NCHUNK = ROWS2D // RPW

_mesh = plsc.VectorSubcoreMesh(core_axis_name="c", subcore_axis_name="s")


def _phase1_body(mem_hbm, idx_hbm, h_hbm, w_hbm, idx_v, rows_v, w_v, pos_v, sem):
  wid = lax.axis_index("s") * NC + lax.axis_index("c")
  rbase = wid * RPW

  pltpu.sync_copy(idx_hbm.at[pl.ds(rbase, RPW)], idx_v)
  for j in range(NSUB):
    pltpu.async_copy(mem_hbm.at[idx_v.at[j]], rows_v, sem).wait()
    pltpu.sync_copy(rows_v, h_hbm.at[pl.ds(wid * BPW + j * SUB, SUB)])

  @pl.when(wid == DEDUP_WID)
  def _dedup():
    iota16 = lax.iota(jnp.int32, 16)

    def pass1_chunk(c, carry):
      pltpu.sync_copy(idx_hbm.at[pl.ds(c * RPW, RPW)], idx_v)

      def vec_body(v, carry2):
        r = v // 8
        lane0 = (v % 8) * 16
        k = idx_v[r, pl.ds(lane0, 16)]
        b = iota16 + (c * (RPW * SUB) + v * 16)
        plsc.store_scatter(pos_v, [k], b)
        g = plsc.load_gather(pos_v, [k])
        has_dup = jnp.any(g != b)

        def fix(carry3):
          def lane_body(j, carry4):
            plsc.store_scatter(pos_v, [k], b, mask=iota16 == j)
            return carry4
          return lax.fori_loop(0, 16, lane_body, carry3)

        return lax.cond(has_dup, fix, lambda carry3: carry3, carry2)

      return lax.fori_loop(0, RPW * SUB // 16, vec_body, carry)

    lax.fori_loop(0, NCHUNK, pass1_chunk, 0)

    def pass2_chunk(c, carry):
      pltpu.sync_copy(idx_hbm.at[pl.ds(c * RPW, RPW)], idx_v)

      def vec_body(v, carry2):
        r = v // 8
        lane0 = (v % 8) * 16
        k = idx_v[r, pl.ds(lane0, 16)]
        w_v[r, pl.ds(lane0, 16)] = plsc.load_gather(pos_v, [k])
        return carry2

      lax.fori_loop(0, RPW * SUB // 16, vec_body, carry)
      pltpu.sync_copy(w_v, w_hbm.at[pl.ds(c * RPW, RPW)])
      return carry

    lax.fori_loop(0, NCHUNK, pass2_chunk, 0)


_phase1 = functools.partial(
    pl.kernel,
    out_type=(
        jax.ShapeDtypeStruct((B, D), jnp.float32),
        jax.ShapeDtypeStruct((ROWS2D, SUB), jnp.int32),
    ),
    mesh=_mesh,
    scratch_types=[
        pltpu.VMEM((RPW, SUB), jnp.int32),
        pltpu.VMEM((SUB, D), jnp.float32),
        pltpu.VMEM((RPW, SUB), jnp.int32),
        pltpu.VMEM((M,), jnp.int32),
        pltpu.SemaphoreType.DMA,
    ],
)(_phase1_body)


def _gru_body(val_ref, h_ref, wih_ref, whh_ref, bih_ref, bhh_ref, out_ref):
  x = val_ref[...]
  h = h_ref[...]
  dn = (((1,), (1,)), ((), ()))
  gi = lax.dot_general(x, wih_ref[...], dn,
                       preferred_element_type=jnp.float32) + bih_ref[...]
  gh = lax.dot_general(h, whh_ref[...], dn,
                       preferred_element_type=jnp.float32) + bhh_ref[...]
  r = jax.nn.sigmoid(gi[:, :D] + gh[:, :D])
  z = jax.nn.sigmoid(gi[:, D:2 * D] + gh[:, D:2 * D])
  n = jnp.tanh(gi[:, 2 * D:] + r * gh[:, 2 * D:])
  out_ref[...] = (1.0 - z) * n + z * h


_GRU_BLK = 2048


def _gru(val, h, W_ih, W_hh, b_ih, b_hh):
  grid = (B // _GRU_BLK,)
  return pl.pallas_call(
      _gru_body,
      grid=grid,
      in_specs=[
          pl.BlockSpec((_GRU_BLK, D), lambda i: (i, 0)),
          pl.BlockSpec((_GRU_BLK, D), lambda i: (i, 0)),
          pl.BlockSpec((3 * D, D), lambda i: (0, 0)),
          pl.BlockSpec((3 * D, D), lambda i: (0, 0)),
          pl.BlockSpec((1, 3 * D), lambda i: (0, 0)),
          pl.BlockSpec((1, 3 * D), lambda i: (0, 0)),
      ],
      out_specs=pl.BlockSpec((_GRU_BLK, D), lambda i: (i, 0)),
      out_shape=jax.ShapeDtypeStruct((B, D), jnp.float32),
  )(val, h, W_ih, W_hh, b_ih, b_hh)


def _phase3_body(hnew_hbm, idx_hbm, w_hbm, out_hbm, idx_v, w_v, rows_v, sem):
  wid = lax.axis_index("s") * NC + lax.axis_index("c")
  rbase = wid * RPW
  pltpu.sync_copy(idx_hbm.at[pl.ds(rbase, RPW)], idx_v)
  pltpu.sync_copy(w_hbm.at[pl.ds(rbase, RPW)], w_v)
  for j in range(NSUB):
    pltpu.async_copy(hnew_hbm.at[w_v.at[j]], rows_v, sem).wait()
    pltpu.sync_copy(rows_v, out_hbm.at[idx_v.at[j]])


_phase3 = functools.partial(
    pl.kernel,
    out_type=(),
    mesh=_mesh,
    scratch_types=[
        pltpu.VMEM((RPW, SUB), jnp.int32),
        pltpu.VMEM((RPW, SUB), jnp.int32),
        pltpu.VMEM((SUB, D), jnp.float32),
        pltpu.SemaphoreType.DMA,
    ],
)(_phase3_body)


def kernel(mem, idx, val, W_ih, W_hh, b_ih, b_hh):
  idx2d = idx.astype(jnp.int32).reshape(ROWS2D, SUB)
  h, w2d = _phase1(mem, idx2d)
  h_new = _gru(val, h, W_ih, W_hh, b_ih.reshape(1, 3 * D),
               b_hh.reshape(1, 3 * D))
  out_ref = jax.new_ref(mem)
  _phase3(h_new, idx2d, w2d, out_ref)
  return out_ref[...]


# speedup vs baseline: 1.1742x; 1.1742x over previous
import functools

import jax
import jax.numpy as jnp
from jax import lax
from jax.experimental import pallas as pl
from jax.experimental.pallas import tpu as pltpu
from jax.experimental.pallas import tpu_sc as plsc

M = 100000
D = 128
B = 16384
NC, NS = 2, 16
NW = NC * NS
BPW = B // NW
SUB = 128
NSUB = BPW // SUB
ROWS2D = B // SUB
RPW = ROWS2D // NW
DEDUP_WID = NW - 1
NCHUNK = ROWS2D // RPW

@functools.lru_cache(maxsize=None)
def _get_mesh():
  return plsc.VectorSubcoreMesh(core_axis_name="c", subcore_axis_name="s",
                                num_cores=NC, num_subcores=NS)


def _phase1_body(mem_hbm, idx_hbm, h_hbm, w_hbm, idx_v, rows_v, w_v, pos_v, sem):
  wid = lax.axis_index("s") * NC + lax.axis_index("c")
  rbase = wid * RPW

  pltpu.sync_copy(idx_hbm.at[pl.ds(rbase, RPW)], idx_v)
  for j in range(NSUB):
    pltpu.async_copy(mem_hbm.at[idx_v.at[j]], rows_v, sem).wait()
    pltpu.sync_copy(rows_v, h_hbm.at[pl.ds(wid * BPW + j * SUB, SUB)])

  @pl.when(wid == DEDUP_WID)
  def _dedup():
    iota16 = lax.iota(jnp.int32, 16)

    def pass1_chunk(c, carry):
      pltpu.sync_copy(idx_hbm.at[pl.ds(c * RPW, RPW)], idx_v)

      def vec_body(v, carry2):
        r = v // 8
        lane0 = (v % 8) * 16
        k = idx_v[r, pl.ds(lane0, 16)]
        b = iota16 + (c * (RPW * SUB) + v * 16)
        plsc.store_scatter(pos_v, [k], b)
        g = plsc.load_gather(pos_v, [k])
        has_dup = jnp.any(g != b)

        def fix(carry3):
          def lane_body(j, carry4):
            plsc.store_scatter(pos_v, [k], b, mask=iota16 == j)
            return carry4
          return lax.fori_loop(0, 16, lane_body, carry3)

        return lax.cond(has_dup, fix, lambda carry3: carry3, carry2)

      return lax.fori_loop(0, RPW * SUB // 16, vec_body, carry)

    lax.fori_loop(0, NCHUNK, pass1_chunk, 0)

    def pass2_chunk(c, carry):
      pltpu.sync_copy(idx_hbm.at[pl.ds(c * RPW, RPW)], idx_v)

      def vec_body(v, carry2):
        r = v // 8
        lane0 = (v % 8) * 16
        k = idx_v[r, pl.ds(lane0, 16)]
        w_v[r, pl.ds(lane0, 16)] = plsc.load_gather(pos_v, [k])
        return carry2

      lax.fori_loop(0, RPW * SUB // 16, vec_body, carry)
      pltpu.sync_copy(w_v, w_hbm.at[pl.ds(c * RPW, RPW)])
      return carry

    lax.fori_loop(0, NCHUNK, pass2_chunk, 0)


@functools.lru_cache(maxsize=None)
def _phase1():
  return pl.kernel(
      _phase1_body,
      out_type=(
          jax.ShapeDtypeStruct((B, D), jnp.float32),
          jax.ShapeDtypeStruct((ROWS2D, SUB), jnp.int32),
      ),
      mesh=_get_mesh(),
      compiler_params=pltpu.CompilerParams(needs_layout_passes=False),
      scratch_types=[
          pltpu.VMEM((RPW, SUB), jnp.int32),
          pltpu.VMEM((SUB, D), jnp.float32),
          pltpu.VMEM((RPW, SUB), jnp.int32),
          pltpu.VMEM((M,), jnp.int32),
          pltpu.SemaphoreType.DMA,
      ],
  )


def _gru_body(val_ref, h_ref, wih_ref, whh_ref, bih_ref, bhh_ref, out_ref):
  x = val_ref[...]
  h = h_ref[...]
  dn = (((1,), (1,)), ((), ()))
  gi = lax.dot_general(x, wih_ref[...], dn,
                       preferred_element_type=jnp.float32) + bih_ref[...]
  gh = lax.dot_general(h, whh_ref[...], dn,
                       preferred_element_type=jnp.float32) + bhh_ref[...]
  r = jax.nn.sigmoid(gi[:, :D] + gh[:, :D])
  z = jax.nn.sigmoid(gi[:, D:2 * D] + gh[:, D:2 * D])
  n = jnp.tanh(gi[:, 2 * D:] + r * gh[:, 2 * D:])
  out_ref[...] = (1.0 - z) * n + z * h


_GRU_BLK = 2048


def _gru(val, h, W_ih, W_hh, b_ih, b_hh):
  grid = (B // _GRU_BLK,)
  return pl.pallas_call(
      _gru_body,
      grid=grid,
      in_specs=[
          pl.BlockSpec((_GRU_BLK, D), lambda i: (i, 0)),
          pl.BlockSpec((_GRU_BLK, D), lambda i: (i, 0)),
          pl.BlockSpec((3 * D, D), lambda i: (0, 0)),
          pl.BlockSpec((3 * D, D), lambda i: (0, 0)),
          pl.BlockSpec((1, 3 * D), lambda i: (0, 0)),
          pl.BlockSpec((1, 3 * D), lambda i: (0, 0)),
      ],
      out_specs=pl.BlockSpec((_GRU_BLK, D), lambda i: (i, 0)),
      out_shape=jax.ShapeDtypeStruct((B, D), jnp.float32),
  )(val, h, W_ih, W_hh, b_ih, b_hh)


def _phase3_body(hnew_hbm, idx_hbm, w_hbm, out_hbm, idx_v, w_v, rows_v, sem):
  wid = lax.axis_index("s") * NC + lax.axis_index("c")
  rbase = wid * RPW
  pltpu.sync_copy(idx_hbm.at[pl.ds(rbase, RPW)], idx_v)
  pltpu.sync_copy(w_hbm.at[pl.ds(rbase, RPW)], w_v)
  for j in range(NSUB):
    pltpu.async_copy(hnew_hbm.at[w_v.at[j]], rows_v, sem).wait()
    pltpu.sync_copy(rows_v, out_hbm.at[idx_v.at[j]])


@functools.lru_cache(maxsize=None)
def _phase3():
  return pl.kernel(
      _phase3_body,
      out_type=(),
      mesh=_get_mesh(),
      scratch_types=[
          pltpu.VMEM((RPW, SUB), jnp.int32),
          pltpu.VMEM((RPW, SUB), jnp.int32),
          pltpu.VMEM((SUB, D), jnp.float32),
          pltpu.SemaphoreType.DMA,
      ],
  )


def kernel(mem, idx, val, W_ih, W_hh, b_ih, b_hh):
  idx2d = idx.astype(jnp.int32).reshape(ROWS2D, SUB)
  h, w2d = _phase1()(mem, idx2d)
  h_new = _gru(val, h, W_ih, W_hh, b_ih.reshape(1, 3 * D),
               b_hh.reshape(1, 3 * D))
  out_ref = jax.new_ref(mem)
  _phase3()(h_new, idx2d, w2d, out_ref)
  return out_ref[...]


# speedup vs baseline: 1.5886x; 1.3530x over previous
import functools

import jax
import jax.numpy as jnp
from jax import lax
from jax.experimental import pallas as pl
from jax.experimental.pallas import tpu as pltpu
from jax.experimental.pallas import tpu_sc as plsc

M = 100000
D = 128
B = 16384
NC, NS = 2, 16
NW = NC * NS
BPW = B // NW
SUB = 128
NSUB = BPW // SUB
ROWS2D = B // SUB
RPW = ROWS2D // NW
DEDUP_WID = NW - 1
DCH = 32

@functools.lru_cache(maxsize=None)
def _get_mesh():
  return plsc.VectorSubcoreMesh(core_axis_name="c", subcore_axis_name="s",
                                num_cores=NC, num_subcores=NS)


def _phase1_body(mem_hbm, idx_hbm, h_hbm, w_hbm, idx_v, rows_v, didx_v, dw_v,
                 pos_v, sem):
  wid = lax.axis_index("s") * NC + lax.axis_index("c")
  rbase = wid * RPW

  pltpu.sync_copy(idx_hbm.at[pl.ds(rbase, RPW)], idx_v)
  for j in range(NSUB):
    pltpu.async_copy(mem_hbm.at[idx_v.at[j]], rows_v, sem).wait()
    pltpu.sync_copy(rows_v, h_hbm.at[pl.ds(wid * BPW + j * SUB, SUB)])

  @pl.when(wid == DEDUP_WID)
  def _dedup():
    iota16 = lax.iota(jnp.int32, 16)
    VPC = DCH * SUB // 16

    def pass1_chunk(c, carry):
      pltpu.sync_copy(idx_hbm.at[pl.ds(c * DCH, DCH)], didx_v)

      def vec_body(v, carry2):
        r = v // 8
        lane0 = (v % 8) * 16
        k = didx_v[r, pl.ds(lane0, 16)]
        b = iota16 + (c * (DCH * SUB) + v * 16)
        _, last = plsc.scan_count(k)
        plsc.store_scatter(pos_v, [k], b)
        plsc.store_scatter(pos_v, [k], b, mask=last)
        return carry2

      return lax.fori_loop(0, VPC, vec_body, carry, unroll=8)

    lax.fori_loop(0, ROWS2D // DCH, pass1_chunk, 0)

    def pass2_chunk(c, carry):
      pltpu.sync_copy(idx_hbm.at[pl.ds(c * DCH, DCH)], didx_v)

      def vec_body(v, carry2):
        r = v // 8
        lane0 = (v % 8) * 16
        k = didx_v[r, pl.ds(lane0, 16)]
        dw_v[r, pl.ds(lane0, 16)] = plsc.load_gather(pos_v, [k])
        return carry2

      lax.fori_loop(0, VPC, vec_body, carry, unroll=8)
      pltpu.sync_copy(dw_v, w_hbm.at[pl.ds(c * DCH, DCH)])
      return carry

    lax.fori_loop(0, ROWS2D // DCH, pass2_chunk, 0)


@functools.lru_cache(maxsize=None)
def _phase1():
  return pl.kernel(
      _phase1_body,
      out_type=(
          jax.ShapeDtypeStruct((B, D), jnp.float32),
          jax.ShapeDtypeStruct((ROWS2D, SUB), jnp.int32),
      ),
      mesh=_get_mesh(),
      compiler_params=pltpu.CompilerParams(needs_layout_passes=False),
      scratch_types=[
          pltpu.VMEM((RPW, SUB), jnp.int32),
          pltpu.VMEM((SUB, D), jnp.float32),
          pltpu.VMEM((DCH, SUB), jnp.int32),
          pltpu.VMEM((DCH, SUB), jnp.int32),
          pltpu.VMEM((M,), jnp.int32),
          pltpu.SemaphoreType.DMA,
      ],
  )


def _gru_body(val_ref, h_ref, wih_ref, whh_ref, bih_ref, bhh_ref, out_ref):
  x = val_ref[...]
  h = h_ref[...]
  dn = (((1,), (1,)), ((), ()))
  gi = lax.dot_general(x, wih_ref[...], dn,
                       preferred_element_type=jnp.float32) + bih_ref[...]
  gh = lax.dot_general(h, whh_ref[...], dn,
                       preferred_element_type=jnp.float32) + bhh_ref[...]
  r = jax.nn.sigmoid(gi[:, :D] + gh[:, :D])
  z = jax.nn.sigmoid(gi[:, D:2 * D] + gh[:, D:2 * D])
  n = jnp.tanh(gi[:, 2 * D:] + r * gh[:, 2 * D:])
  out_ref[...] = (1.0 - z) * n + z * h


_GRU_BLK = 2048


def _gru(val, h, W_ih, W_hh, b_ih, b_hh):
  grid = (B // _GRU_BLK,)
  return pl.pallas_call(
      _gru_body,
      grid=grid,
      in_specs=[
          pl.BlockSpec((_GRU_BLK, D), lambda i: (i, 0)),
          pl.BlockSpec((_GRU_BLK, D), lambda i: (i, 0)),
          pl.BlockSpec((3 * D, D), lambda i: (0, 0)),
          pl.BlockSpec((3 * D, D), lambda i: (0, 0)),
          pl.BlockSpec((1, 3 * D), lambda i: (0, 0)),
          pl.BlockSpec((1, 3 * D), lambda i: (0, 0)),
      ],
      out_specs=pl.BlockSpec((_GRU_BLK, D), lambda i: (i, 0)),
      out_shape=jax.ShapeDtypeStruct((B, D), jnp.float32),
  )(val, h, W_ih, W_hh, b_ih, b_hh)


def _phase3_body(hnew_hbm, idx_hbm, w_hbm, out_hbm, idx_v, w_v, rows_v, sem):
  wid = lax.axis_index("s") * NC + lax.axis_index("c")
  rbase = wid * RPW
  pltpu.sync_copy(idx_hbm.at[pl.ds(rbase, RPW)], idx_v)
  pltpu.sync_copy(w_hbm.at[pl.ds(rbase, RPW)], w_v)
  for j in range(NSUB):
    pltpu.async_copy(hnew_hbm.at[w_v.at[j]], rows_v, sem).wait()
    pltpu.sync_copy(rows_v, out_hbm.at[idx_v.at[j]])


@functools.lru_cache(maxsize=None)
def _phase3():
  return pl.kernel(
      _phase3_body,
      out_type=(),
      mesh=_get_mesh(),
      scratch_types=[
          pltpu.VMEM((RPW, SUB), jnp.int32),
          pltpu.VMEM((RPW, SUB), jnp.int32),
          pltpu.VMEM((SUB, D), jnp.float32),
          pltpu.SemaphoreType.DMA,
      ],
  )


def kernel(mem, idx, val, W_ih, W_hh, b_ih, b_hh):
  idx2d = idx.astype(jnp.int32).reshape(ROWS2D, SUB)
  h, w2d = _phase1()(mem, idx2d)
  h_new = _gru(val, h, W_ih, W_hh, b_ih.reshape(1, 3 * D),
               b_hh.reshape(1, 3 * D))
  out_ref = jax.new_ref(mem)
  _phase3()(h_new, idx2d, w2d, out_ref)
  return out_ref[...]
